# baseline (speedup 1.0000x reference)
import functools

import jax
import jax.numpy as jnp
from jax import lax
from jax.experimental import pallas as pl
from jax.experimental.pallas import tpu as pltpu
from jax.experimental.pallas import tpu_sc as plsc

_N = 1000000
_C = 64
_NBINS = 10
_NW = 32
_CH = 512
_NFULL = _N // _CH
_TAIL = _N - _NFULL * _CH
_BASE = _NFULL // _NW
_EXTRA = _NFULL - _BASE * _NW


def _sc_body(logits_hbm, targets_hbm, out_hbm, xbuf, tbuf, bins):
    wid = lax.axis_index("s") * 2 + lax.axis_index("c")
    zeros16 = jnp.zeros((16,), jnp.float32)
    bins[pl.ds(0, 16)] = zeros16
    bins[pl.ds(16, 16)] = zeros16
    bins[pl.ds(32, 16)] = zeros16

    iota16 = lax.iota(jnp.int32, 16)
    ones16 = jnp.full((16,), 1.0, jnp.float32)

    def group_body(g, carry):
        fidx = (iota16 + g * 16) * _C
        s = jnp.zeros((16,), jnp.float32)
        mx = jnp.zeros((16,), jnp.float32)
        for c in range(_C):
            e = jnp.exp(plsc.load_gather(xbuf, [fidx + c]))
            s = s + e
            mx = jnp.maximum(mx, e)
        tv = tbuf[pl.ds(g * 16, 16)]
        et = jnp.exp(plsc.load_gather(xbuf, [fidx + tv]))
        acc = jnp.where(et == mx, 1.0, 0.0)
        conf = mx / s
        b = jnp.minimum((conf * jnp.float32(_NBINS)).astype(jnp.int32),
                        _NBINS - 1)
        plsc.addupdate_scatter(bins, [b], ones16)
        plsc.addupdate_scatter(bins, [b + 16], conf)
        plsc.addupdate_scatter(bins, [b + 32], acc)
        return carry

    start = jnp.where(wid == 0, 0, _BASE * wid + _EXTRA)
    nch = jnp.where(wid == 0, _BASE + _EXTRA, _BASE)

    def chunk_body(j, carry):
        row0 = (start + j) * _CH
        pltpu.sync_copy(logits_hbm.at[pl.ds(row0 * _C, _CH * _C)], xbuf)
        pltpu.sync_copy(targets_hbm.at[pl.ds(row0, _CH)], tbuf)
        lax.fori_loop(0, _CH // 16, group_body, carry)
        return carry

    lax.fori_loop(0, nch, chunk_body, 0)

    @pl.when(wid == _NW - 1)
    def _tail():
        row0 = _NFULL * _CH
        pltpu.sync_copy(logits_hbm.at[pl.ds(row0 * _C, _TAIL * _C)],
                        xbuf.at[pl.ds(0, _TAIL * _C)])
        pltpu.sync_copy(targets_hbm.at[pl.ds(row0, _TAIL)],
                        tbuf.at[pl.ds(0, _TAIL)])
        lax.fori_loop(0, _TAIL // 16, group_body, 0)

    pltpu.sync_copy(bins, out_hbm.at[wid])


def _finish_kernel(s_ref, o_ref):
    tot = jnp.sum(s_ref[...], axis=0, keepdims=True)
    cnt = tot[0:1, 0:_NBINS]
    sc = tot[0:1, 16:16 + _NBINS]
    sa = tot[0:1, 32:32 + _NBINS]
    safe = jnp.maximum(cnt, 1.0)
    contrib = jnp.where(
        cnt > 0.0,
        jnp.abs(sc / safe - sa / safe) * (cnt / jnp.float32(_N)),
        0.0,
    )
    o_ref[...] = jnp.sum(contrib, axis=1, keepdims=True)


def kernel(logits, targets):
    sc_fn = pl.kernel(
        _sc_body,
        out_type=jax.ShapeDtypeStruct((_NW, 48), jnp.float32),
        mesh=plsc.VectorSubcoreMesh(core_axis_name="c", subcore_axis_name="s"),
        compiler_params=pltpu.CompilerParams(needs_layout_passes=False),
        scratch_types=[
            pltpu.VMEM((_CH * _C,), jnp.float32),
            pltpu.VMEM((_CH,), jnp.int32),
            pltpu.VMEM((48,), jnp.float32),
        ],
    )
    stats = sc_fn(logits.reshape(-1), targets)
    ece = pl.pallas_call(
        _finish_kernel,
        out_shape=jax.ShapeDtypeStruct((1, 1), jnp.float32),
    )(stats)
    return ece.reshape(1)


# speedup vs baseline: 1.4462x; 1.4462x over previous
import functools

import jax
import jax.numpy as jnp
from jax import lax
from jax.experimental import pallas as pl
from jax.experimental.pallas import tpu as pltpu
from jax.experimental.pallas import tpu_sc as plsc

_N = 1000000
_C = 64
_NBINS = 10
_NW = 32
_CH = 512
_NFULL = _N // _CH
_TAIL = _N - _NFULL * _CH
_BASE = _NFULL // _NW
_EXTRA = _NFULL - _BASE * _NW


def _sc_body(logits_hbm, targets_hbm, out_hbm, xbuf, tbuf, bins):
    wid = lax.axis_index("s") * 2 + lax.axis_index("c")
    zeros16 = jnp.zeros((16,), jnp.float32)
    bins[pl.ds(0, 16)] = zeros16
    bins[pl.ds(16, 16)] = zeros16
    bins[pl.ds(32, 16)] = zeros16

    iota16 = lax.iota(jnp.int32, 16)
    ones16 = jnp.full((16,), 1.0, jnp.float32)

    def group_body(g, carry):
        fidx = (iota16 + g * 16) * _C
        s = jnp.zeros((16,), jnp.float32)
        mx = jnp.zeros((16,), jnp.float32)
        for c in range(_C):
            col = (iota16 + c) & (_C - 1)
            e = jnp.exp(plsc.load_gather(xbuf, [fidx + col]))
            s = s + e
            mx = jnp.maximum(mx, e)
        tv = tbuf[pl.ds(g * 16, 16)]
        et = jnp.exp(plsc.load_gather(xbuf, [fidx + tv]))
        acc = jnp.where(et == mx, 1.0, 0.0)
        conf = mx / s
        b = jnp.minimum((conf * jnp.float32(_NBINS)).astype(jnp.int32),
                        _NBINS - 1)
        plsc.addupdate_scatter(bins, [b], ones16)
        plsc.addupdate_scatter(bins, [b + 16], conf)
        plsc.addupdate_scatter(bins, [b + 32], acc)
        return carry

    start = jnp.where(wid == 0, 0, _BASE * wid + _EXTRA)
    nch = jnp.where(wid == 0, _BASE + _EXTRA, _BASE)

    def chunk_body(j, carry):
        row0 = (start + j) * _CH
        pltpu.sync_copy(logits_hbm.at[pl.ds(row0 * _C, _CH * _C)], xbuf)
        pltpu.sync_copy(targets_hbm.at[pl.ds(row0, _CH)], tbuf)
        lax.fori_loop(0, _CH // 16, group_body, carry)
        return carry

    lax.fori_loop(0, nch, chunk_body, 0)

    @pl.when(wid == _NW - 1)
    def _tail():
        row0 = _NFULL * _CH
        pltpu.sync_copy(logits_hbm.at[pl.ds(row0 * _C, _TAIL * _C)],
                        xbuf.at[pl.ds(0, _TAIL * _C)])
        pltpu.sync_copy(targets_hbm.at[pl.ds(row0, _TAIL)],
                        tbuf.at[pl.ds(0, _TAIL)])
        lax.fori_loop(0, _TAIL // 16, group_body, 0)

    pltpu.sync_copy(bins, out_hbm.at[wid])


def _finish_kernel(s_ref, o_ref):
    tot = jnp.sum(s_ref[...], axis=0, keepdims=True)
    cnt = tot[0:1, 0:_NBINS]
    sc = tot[0:1, 16:16 + _NBINS]
    sa = tot[0:1, 32:32 + _NBINS]
    safe = jnp.maximum(cnt, 1.0)
    contrib = jnp.where(
        cnt > 0.0,
        jnp.abs(sc / safe - sa / safe) * (cnt / jnp.float32(_N)),
        0.0,
    )
    o_ref[...] = jnp.sum(contrib, axis=1, keepdims=True)


def kernel(logits, targets):
    sc_fn = pl.kernel(
        _sc_body,
        out_type=jax.ShapeDtypeStruct((_NW, 48), jnp.float32),
        mesh=plsc.VectorSubcoreMesh(core_axis_name="c", subcore_axis_name="s"),
        compiler_params=pltpu.CompilerParams(needs_layout_passes=False),
        scratch_types=[
            pltpu.VMEM((_CH * _C,), jnp.float32),
            pltpu.VMEM((_CH,), jnp.int32),
            pltpu.VMEM((48,), jnp.float32),
        ],
    )
    stats = sc_fn(logits.reshape(-1), targets)
    ece = pl.pallas_call(
        _finish_kernel,
        out_shape=jax.ShapeDtypeStruct((1, 1), jnp.float32),
    )(stats)
    return ece.reshape(1)
